# Initial kernel scaffold; baseline (speedup 1.0000x reference)
import jax
import jax.numpy as jnp
from jax.experimental import pallas as pl


def kernel(x, edge_index, eps, W, b):
    raise NotImplementedError("write your pallas kernel here")


# speedup vs baseline: 5.1771x; 5.1771x over previous
import functools

import jax
import jax.numpy as jnp
from jax import lax
from jax.experimental import pallas as pl
from jax.experimental.pallas import tpu as pltpu
from jax.experimental.pallas import tpu_sc as plsc

N = 10000
K = 32
C = 128

NC = 2
NS = 16
NW = NC * NS
CH = 64
NCH = 5
NPT = CH * NCH
NPAD = NW * NPT


def _sc_agg_kernel(x_hbm, idx_hbm, out_hbm, idx_v, acc, sem):
    wid = lax.axis_index("s") * NC + lax.axis_index("c")
    base = wid * NPT

    pltpu.sync_copy(idx_hbm.at[wid], idx_v)

    descs = []
    for c in range(NCH):
        descs.append(
            pltpu.async_copy(x_hbm.at[idx_v.at[0, c]],
                             acc.at[pl.ds(c * CH, CH)], sem))
    for d in descs:
        d.wait()

    def body(k, carry):
        ds = []
        for c in range(NCH):
            ds.append(
                pltpu.async_copy(x_hbm.at[idx_v.at[k, c]],
                                 acc.at[pl.ds(c * CH, CH)], sem, add=True))
        for d in ds:
            d.wait()
        return carry

    lax.fori_loop(1, K, body, 0, unroll=False)

    pltpu.sync_copy(acc, out_hbm.at[pl.ds(base, NPT)])


def _sc_agg(x2d, idx_arr):
    mesh = plsc.VectorSubcoreMesh(core_axis_name="c", subcore_axis_name="s")
    f = pl.kernel(
        _sc_agg_kernel,
        out_type=jax.ShapeDtypeStruct((NPAD, C), jnp.float32),
        mesh=mesh,
        scratch_types=[
            pltpu.VMEM((K, NCH, CH), jnp.int32),
            pltpu.VMEM((NPT, C), jnp.float32),
            pltpu.SemaphoreType.DMA,
        ],
    )
    return f(x2d, idx_arr)


ROWS = 1000


def _mlp_body(scale_ref, x_ref, agg_ref, w_ref, b_ref, o_ref):
    h = scale_ref[0] * x_ref[...] + agg_ref[...]
    y = lax.dot_general(h, w_ref[...], (((1,), (1,)), ((), ())),
                        preferred_element_type=jnp.float32)
    o_ref[...] = jnp.maximum(y + b_ref[...], 0.0)


def _mlp(scale, x2d, agg, w, b2d):
    grid = (N // ROWS,)
    return pl.pallas_call(
        _mlp_body,
        grid=grid,
        in_specs=[
            pl.BlockSpec(memory_space=pltpu.SMEM),
            pl.BlockSpec((ROWS, C), lambda i: (i, 0)),
            pl.BlockSpec((ROWS, C), lambda i: (i, 0)),
            pl.BlockSpec((C, C), lambda i: (0, 0)),
            pl.BlockSpec((1, C), lambda i: (0, 0)),
        ],
        out_specs=pl.BlockSpec((ROWS, C), lambda i: (i, 0)),
        out_shape=jax.ShapeDtypeStruct((N, C), jnp.float32),
    )(scale, x2d, agg, w, b2d)


def kernel(x, edge_index, eps, W, b):
    x2d = x[0]
    idx = edge_index[0, 0]
    idx_pad = jnp.pad(idx, ((0, NPAD - N), (0, 0)))
    idx_arr = idx_pad.reshape(NW, NCH, CH, K).transpose(0, 3, 1, 2)

    agg = _sc_agg(x2d, idx_arr)[:N]

    scale = (1.0 + eps).reshape(1)
    out2d = _mlp(scale, x2d, agg, W, b.reshape(1, C))
    return out2d.reshape(1, N, C)
